# baseline (speedup 1.0000x reference)
import math

import jax
import jax.numpy as jnp
from jax import lax
from jax.experimental import pallas as pl
from jax.experimental.pallas import tpu as pltpu

_B, _Q, _S, _T, _EX = 8, 128, 32, 64, 64
_D, _DI, _DFF, _H = 512, 64, 2048, 8
_DH = _D // _H
_STAT_K, _TOKEN_K = 4, 8
_NEG = -1e6
_F32 = jnp.float32


def _dot(a, b):
    return lax.dot_general(a.astype(b.dtype), b, (((1,), (0,)), ((), ())),
                           preferred_element_type=_F32)


def _dot_t(a, b):
    return lax.dot_general(a.astype(b.dtype), b, (((1,), (1,)), ((), ())),
                           preferred_element_type=_F32)


def _layer_norm(x, g, b):
    m = jnp.mean(x, axis=-1, keepdims=True)
    c = x - m
    v = jnp.mean(c * c, axis=-1, keepdims=True)
    return c * lax.rsqrt(v + 1e-5) * g + b


def _kth_threshold(s, k, axis):
    t = jnp.max(s, axis=axis, keepdims=True)
    for _ in range(k - 1):
        t = jnp.max(jnp.where(s < t, s, -jnp.inf), axis=axis, keepdims=True)
    return jnp.maximum(t, _NEG)


def _softmax_ax(x, axis):
    m = jnp.max(x, axis=axis, keepdims=True)
    e = jnp.exp(x - m)
    return e / jnp.sum(e, axis=axis, keepdims=True)


def _attn1(Qp, Kp, Vp, valid, nq, nk):
    scale = 1.0 / math.sqrt(_DH)
    kidx = lax.broadcasted_iota(jnp.int32, (nq, nk), 1)
    ones = jnp.ones((nk, 8), dtype=_F32)
    fill = jnp.where(valid == 0, 0.0, _NEG)
    mask = kidx < valid
    outs = []
    for h in range(_H):
        sl = slice(h * _DH, (h + 1) * _DH)
        s = _dot_t(Qp[:, sl], Kp[:, sl]) * scale
        e = jnp.exp(jnp.where(mask, s, fill))
        av = jnp.concatenate([Vp[:, sl], ones], axis=1)
        o = _dot(e, av)
        outs.append(o[:, :_DH] * (1.0 / o[:, _DH:_DH + 1]))
    return jnp.concatenate(outs, axis=-1)


def _blk_fused(dec_ref, exv_ref, stat_ref,
               x_ref, int_ref, ex_ref, sk_ref, tk_ref, val_ref,
               mwq, mwk, mwv, mwo, g1, b1, cwq, cwk, cwv, cwo,
               wqs, wqt, wks, wkt, wv, wo,
               gt, w1, fb1, w2, fb2, g2, bb2, g3, bb3,
               out_ref,
               ss_s, ts_s, x1_s, exo_s):
    b = pl.program_id(0)
    bf = jnp.minimum(b, _B - 1)
    par = lax.rem(b, 2)
    opar = 1 - par
    scale = 1.0 / math.sqrt(_D)

    xf = x_ref[0]
    y = _attn1(_dot(xf, mwq[...]), _dot(xf, mwk[...]),
               _dot(xf, mwv[...]), dec_ref[bf], _Q, _Q)
    x1 = _layer_norm(xf + _dot(y, mwo[...]), g1[...], b1[...])
    x1_s[par] = x1
    qc = jnp.concatenate([x1, int_ref[0]], axis=-1)
    exf = ex_ref[0]
    co = _attn1(_dot(qc, cwq[...]), _dot(exf, cwk[...]),
                _dot(exf, cwv[...]), exv_ref[bf], _Q, _EX)
    exo_s[par] = _dot(co, cwo[...])

    qs = _dot(qc, wqs[...])
    ks = _dot(sk_ref[0], wks[...])
    sst = _dot_t(ks, qs) * scale
    gidx = lax.broadcasted_iota(jnp.int32, (_S, _Q), 0)
    ss_s[par] = jnp.where(gidx < stat_ref[bf], sst, _NEG)
    qt = _dot(qc, wqt[...])
    qt2 = _dot_t(qt, wkt[...])
    ts_s[par] = _dot_t(tk_ref[0], qt2) * scale

    ssT = ss_s[opar]
    swT = _softmax_ax(
        jnp.where(ssT >= _kth_threshold(ssT, _STAT_K, 0), ssT, _NEG), 0)
    ts3 = ts_s[opar].reshape(_S, _T, _Q)
    tw3 = _softmax_ax(
        jnp.where(ts3 >= _kth_threshold(ts3, _TOKEN_K, 1), ts3, _NEG), 1)
    comb = (swT[:, None, :] * tw3).reshape(_S * _T, _Q)
    ctx = lax.dot_general(comb, val_ref[0], (((0,), (0,)), ((), ())),
                          preferred_element_type=_F32)
    sel = _dot(_dot(ctx, wv[...]), wo[...])
    x1c = x1_s[opar]
    exoc = exo_s[opar]
    gw = gt[...]
    logit = (jnp.sum(sel * gw[:, :_D], axis=-1, keepdims=True)
             + jnp.sum(exoc * gw[:, _D:], axis=-1, keepdims=True))
    g = jax.nn.sigmoid(logit)
    x2 = _layer_norm(x1c + g * sel + (1.0 - g) * exoc, g2[...], bb2[...])
    hh = jnp.maximum(_dot(x2, w1[...]) + fb1[...], 0.0)
    ff = _dot(hh, w2[...]) + fb2[...]
    out_ref[0] = _layer_norm(x2 + ff, g3[...], bb3[...])


def _full(shape):
    n = len(shape)
    return pl.BlockSpec(tuple(shape), lambda b, *_: (0,) * n)


def _lead(shape):
    n = len(shape) - 1
    return pl.BlockSpec((1,) + tuple(shape[1:]),
                        lambda b, *_: (jnp.minimum(b, _B - 1),) + (0,) * n)


def _lag(shape):
    n = len(shape) - 1
    return pl.BlockSpec((1,) + tuple(shape[1:]),
                        lambda b, *_: (jnp.maximum(b - 1, 0),) + (0,) * n)


def kernel(x, intent, stat_keys, token_keys, values, exemplar, params,
           dec_valid_lens, stat_valid_lens, ex_valid_lens):
    P = params
    dec = dec_valid_lens.astype(jnp.int32)
    stv = stat_valid_lens.astype(jnp.int32)
    exv = ex_valid_lens.astype(jnp.int32)
    tk = token_keys.reshape(_B, _S * _T, _D)
    vals = values.reshape(_B, _S * _T, _D)
    r = lambda a, n: a.reshape(1, n)

    weights = [P['ma_Wq'], P['ma_Wk'], P['ma_Wv'], P['ma_Wo'],
               r(P['ln1_g'], _D), r(P['ln1_b'], _D),
               P['ca_Wq'], P['ca_Wk'], P['ca_Wv'], P['ca_Wo'],
               P['sa_Wqs'], P['sa_Wqt'], P['sa_Wks'], P['sa_Wkt'],
               P['sa_Wv'], P['sa_Wo'], P['gate_W'].reshape(1, 2 * _D),
               P['ffn_W1'], r(P['ffn_b1'], _DFF),
               P['ffn_W2'], r(P['ffn_b2'], _D),
               r(P['ln2_g'], _D), r(P['ln2_b'], _D),
               r(P['ln3_g'], _D), r(P['ln3_b'], _D)]
    grid_spec = pltpu.PrefetchScalarGridSpec(
        num_scalar_prefetch=3,
        grid=(_B + 1,),
        in_specs=[_lead(x.shape), _lead(intent.shape), _lead(exemplar.shape),
                  _lead(stat_keys.shape), _lead(tk.shape), _lag(vals.shape),
                  *[_full(w.shape) for w in weights]],
        out_specs=(_lag((_B, _Q, _D)),),
        scratch_shapes=[pltpu.VMEM((2, _S, _Q), _F32),
                        pltpu.VMEM((2, _S * _T, _Q), _F32),
                        pltpu.VMEM((2, _Q, _D), _F32),
                        pltpu.VMEM((2, _Q, _D), _F32)],
    )
    (out,) = pl.pallas_call(
        _blk_fused,
        grid_spec=grid_spec,
        out_shape=(jax.ShapeDtypeStruct((_B, _Q, _D), _F32),),
    )(dec, exv, stv, x, intent, exemplar, stat_keys, tk, vals, *weights)
    return out


# speedup vs baseline: 1.1284x; 1.1284x over previous
import math

import jax
import jax.numpy as jnp
from jax import lax
from jax.experimental import pallas as pl
from jax.experimental.pallas import tpu as pltpu

_B, _Q, _S, _T, _EX = 8, 128, 32, 64, 64
_D, _DI, _DFF, _H = 512, 64, 2048, 8
_DH = _D // _H
_STAT_K, _TOKEN_K = 4, 8
_NEG = -1e6
_F32 = jnp.float32
_BB = 8


def _dot(a, b):
    return lax.dot_general(a.astype(b.dtype), b, (((1,), (0,)), ((), ())),
                           preferred_element_type=_F32)


def _dot_t(a, b):
    return lax.dot_general(a.astype(b.dtype), b, (((1,), (1,)), ((), ())),
                           preferred_element_type=_F32)


def _layer_norm(x, g, b):
    m = jnp.mean(x, axis=-1, keepdims=True)
    c = x - m
    v = jnp.mean(c * c, axis=-1, keepdims=True)
    return c * lax.rsqrt(v + 1e-5) * g + b


def _kth_threshold(s, k, axis):
    t = jnp.max(s, axis=axis, keepdims=True)
    for _ in range(k - 1):
        t = jnp.max(jnp.where(s < t, s, -jnp.inf), axis=axis, keepdims=True)
    return jnp.maximum(t, _NEG)


def _softmax_ax(x, axis):
    m = jnp.max(x, axis=axis, keepdims=True)
    e = jnp.exp(x - m)
    return e / jnp.sum(e, axis=axis, keepdims=True)


def _attn_block(Qp, Kp, Vp, valid_ref, base, nq, nk):
    scale = 1.0 / math.sqrt(_DH)
    kidx = lax.broadcasted_iota(jnp.int32, (nq, nk), 1)
    ones = jnp.ones((nk, _H), dtype=_F32)
    exp_m = (lax.broadcasted_iota(jnp.int32, (_H, _D), 1) // _DH
             == lax.broadcasted_iota(jnp.int32, (_H, _D), 0)).astype(_F32)
    blocks = []
    for i in range(_BB):
        valid = valid_ref[base + i]
        fill = jnp.where(valid == 0, 0.0, _NEG)
        mask = kidx < valid
        qr = slice(i * nq, (i + 1) * nq)
        kr = slice(i * nk, (i + 1) * nk)
        outs = []
        dens = []
        for h in range(_H):
            sl = slice(h * _DH, (h + 1) * _DH)
            s = _dot_t(Qp[qr, sl], Kp[kr, sl]) * scale
            e = jnp.exp(jnp.where(mask, s, fill))
            av = jnp.concatenate([Vp[kr, sl], ones], axis=1)
            o = _dot(e, av)
            outs.append(o[:, :_DH])
            dens.append(o[:, _DH + h:_DH + h + 1])
        den = jnp.concatenate(dens, axis=1)
        dfull = _dot(1.0 / den, exp_m)
        blocks.append(jnp.concatenate(outs, axis=-1) * dfull)
    return jnp.concatenate(blocks, axis=0)


def _blk_front(dec_ref, exv_ref, x_ref, int_ref, ex_ref,
               mwq, mwk, mwv, mwo, g1, b1, cwq, cwk, cwv, cwo,
               x1_ref, qc_ref, exo_ref):
    base = pl.program_id(0) * _BB
    xf = x_ref[...].reshape(_BB * _Q, _D)
    y = _attn_block(_dot(xf, mwq[...]), _dot(xf, mwk[...]),
                    _dot(xf, mwv[...]), dec_ref, base, _Q, _Q)
    x1f = _layer_norm(xf + _dot(y, mwo[...]), g1[...], b1[...])
    x1_ref[...] = x1f.reshape(_BB, _Q, _D)
    qcf = jnp.concatenate(
        [x1f, int_ref[...].reshape(_BB * _Q, _DI)], axis=-1)
    qc_ref[...] = qcf.reshape(_BB, _Q, _D + _DI)
    exf = ex_ref[...].reshape(_BB * _EX, _D)
    co = _attn_block(_dot(qcf, cwq[...]), _dot(exf, cwk[...]),
                     _dot(exf, cwv[...]), exv_ref, base, _Q, _EX)
    exo_ref[...] = _dot(co, cwo[...]).reshape(_BB, _Q, _D)


def _blk_back(stat_ref, qc_ref, sk_ref, tk_ref, val_ref, x1_ref, exo_ref,
              wqs, wqt, wks, wkt, wv, wo,
              gt, w1, b1, w2, b2, g2, bb2, g3, bb3,
              out_ref, ss_s, ts_s):
    b = pl.program_id(0)
    par = lax.rem(b, 2)
    opar = 1 - par
    scale = 1.0 / math.sqrt(_D)

    ssT = ss_s[opar]
    swT = _softmax_ax(
        jnp.where(ssT >= _kth_threshold(ssT, _STAT_K, 0), ssT, _NEG), 0)
    ts3 = ts_s[opar].reshape(_S, _T, _Q)
    tw3 = _softmax_ax(
        jnp.where(ts3 >= _kth_threshold(ts3, _TOKEN_K, 1), ts3, _NEG), 1)
    comb = (swT[:, None, :] * tw3).reshape(_S * _T, _Q)
    ctx = lax.dot_general(comb, val_ref[0], (((0,), (0,)), ((), ())),
                          preferred_element_type=_F32)
    sel = _dot(_dot(ctx, wv[...]), wo[...])
    x1 = x1_ref[0]
    exo = exo_ref[0]
    gw = gt[...]
    logit = (jnp.sum(sel * gw[:, :_D], axis=-1, keepdims=True)
             + jnp.sum(exo * gw[:, _D:], axis=-1, keepdims=True))
    g = jax.nn.sigmoid(logit)
    x2 = _layer_norm(x1 + g * sel + (1.0 - g) * exo, g2[...], bb2[...])
    hh = jnp.maximum(_dot(x2, w1[...]) + b1[...], 0.0)
    ff = _dot(hh, w2[...]) + b2[...]
    out_ref[0] = _layer_norm(x2 + ff, g3[...], bb3[...])

    qc = qc_ref[0]
    qs = _dot(qc, wqs[...])
    ks = _dot(sk_ref[0], wks[...])
    sst = _dot_t(ks, qs) * scale
    gidx = lax.broadcasted_iota(jnp.int32, (_S, _Q), 0)
    ss_s[par] = jnp.where(gidx < stat_ref[jnp.minimum(b, _B - 1)], sst, _NEG)
    qt = _dot(qc, wqt[...])
    qt2 = _dot_t(qt, wkt[...])
    ts_s[par] = _dot_t(tk_ref[0], qt2) * scale


def _batched(shape, nb):
    n = len(shape) - 1
    return pl.BlockSpec((nb,) + tuple(shape[1:]),
                        lambda b, *_: (b,) + (0,) * n)


def _full(shape):
    n = len(shape)
    return pl.BlockSpec(tuple(shape), lambda b, *_: (0,) * n)


def _lead(shape):
    n = len(shape) - 1
    return pl.BlockSpec((1,) + tuple(shape[1:]),
                        lambda b, *_: (jnp.minimum(b, _B - 1),) + (0,) * n)


def _lag(shape):
    n = len(shape) - 1
    return pl.BlockSpec((1,) + tuple(shape[1:]),
                        lambda b, *_: (jnp.maximum(b - 1, 0),) + (0,) * n)


def _call(body, scalars, arrays, out_shapes, nb=1):
    in_specs = [_batched(a.shape, nb) if flag else _full(a.shape)
                for a, flag in arrays]
    grid_spec = pltpu.PrefetchScalarGridSpec(
        num_scalar_prefetch=len(scalars),
        grid=(_B // nb,),
        in_specs=in_specs,
        out_specs=tuple(_batched(s, nb) for s in out_shapes),
    )
    return pl.pallas_call(
        body,
        grid_spec=grid_spec,
        out_shape=tuple(jax.ShapeDtypeStruct(s, _F32) for s in out_shapes),
    )(*scalars, *(a for a, _ in arrays))


def kernel(x, intent, stat_keys, token_keys, values, exemplar, params,
           dec_valid_lens, stat_valid_lens, ex_valid_lens):
    P = params
    dec = dec_valid_lens.astype(jnp.int32)
    stv = stat_valid_lens.astype(jnp.int32)
    exv = ex_valid_lens.astype(jnp.int32)
    tk = token_keys.reshape(_B, _S * _T, _D)
    vals = values.reshape(_B, _S * _T, _D)
    r = lambda a, n: a.reshape(1, n)

    x1, qc, exo = _call(
        _blk_front, (dec, exv),
        [(x, True), (intent, True), (exemplar, True),
         (P['ma_Wq'], False), (P['ma_Wk'], False),
         (P['ma_Wv'], False), (P['ma_Wo'], False),
         (r(P['ln1_g'], _D), False), (r(P['ln1_b'], _D), False),
         (P['ca_Wq'], False), (P['ca_Wk'], False),
         (P['ca_Wv'], False), (P['ca_Wo'], False)],
        [(_B, _Q, _D), (_B, _Q, _D + _DI), (_B, _Q, _D)], nb=_BB)

    weights = [P['sa_Wqs'], P['sa_Wqt'], P['sa_Wks'], P['sa_Wkt'],
               P['sa_Wv'], P['sa_Wo'], P['gate_W'].reshape(1, 2 * _D),
               P['ffn_W1'], r(P['ffn_b1'], _DFF),
               P['ffn_W2'], r(P['ffn_b2'], _D),
               r(P['ln2_g'], _D), r(P['ln2_b'], _D),
               r(P['ln3_g'], _D), r(P['ln3_b'], _D)]
    grid_spec = pltpu.PrefetchScalarGridSpec(
        num_scalar_prefetch=1,
        grid=(_B + 1,),
        in_specs=[_lead(qc.shape), _lead(stat_keys.shape), _lead(tk.shape),
                  _lag(vals.shape), _lag(x1.shape), _lag(exo.shape),
                  *[_full(w.shape) for w in weights]],
        out_specs=(_lag((_B, _Q, _D)),),
        scratch_shapes=[pltpu.VMEM((2, _S, _Q), _F32),
                        pltpu.VMEM((2, _S * _T, _Q), _F32)],
    )
    (out,) = pl.pallas_call(
        _blk_back,
        grid_spec=grid_spec,
        out_shape=(jax.ShapeDtypeStruct((_B, _Q, _D), _F32),),
    )(stv, qc, stat_keys, tk, vals, x1, exo, *weights)
    return out


# speedup vs baseline: 1.1676x; 1.0347x over previous
import math

import jax
import jax.numpy as jnp
from jax import lax
from jax.experimental import pallas as pl
from jax.experimental.pallas import tpu as pltpu

_B, _Q, _S, _T, _EX = 8, 128, 32, 64, 64
_D, _DI, _DFF, _H = 512, 64, 2048, 8
_DH = _D // _H
_STAT_K, _TOKEN_K = 4, 8
_NEG = -1e6
_F32 = jnp.float32
_BB = 8


def _dot(a, b):
    return lax.dot_general(a.astype(b.dtype), b, (((1,), (0,)), ((), ())),
                           preferred_element_type=_F32)


def _dot_t(a, b):
    return lax.dot_general(a.astype(b.dtype), b, (((1,), (1,)), ((), ())),
                           preferred_element_type=_F32)


def _layer_norm(x, g, b):
    m = jnp.mean(x, axis=-1, keepdims=True)
    c = x - m
    v = jnp.mean(c * c, axis=-1, keepdims=True)
    return c * lax.rsqrt(v + 1e-5) * g + b


def _kth_threshold(s, k, axis):
    t = jnp.max(s, axis=axis, keepdims=True)
    for _ in range(k - 1):
        t = jnp.max(jnp.where(s < t, s, -jnp.inf), axis=axis, keepdims=True)
    return jnp.maximum(t, _NEG)


def _softmax_ax(x, axis):
    m = jnp.max(x, axis=axis, keepdims=True)
    e = jnp.exp(x - m)
    return e / jnp.sum(e, axis=axis, keepdims=True)


def _attn_block(Qp, Kp, Vp, valid_ref, base, nq, nk):
    scale = 1.0 / math.sqrt(_DH)
    kidx = lax.broadcasted_iota(jnp.int32, (nq, nk), 1)
    ones = jnp.ones((nk, 8), dtype=_F32)
    blocks = []
    for i in range(_BB):
        valid = valid_ref[base + i]
        fill = jnp.where(valid == 0, 0.0, _NEG)
        mask = kidx < valid
        qr = slice(i * nq, (i + 1) * nq)
        kr = slice(i * nk, (i + 1) * nk)
        outs = []
        for h in range(_H):
            sl = slice(h * _DH, (h + 1) * _DH)
            s = _dot_t(Qp[qr, sl], Kp[kr, sl]) * scale
            e = jnp.exp(jnp.where(mask, s, fill))
            av = jnp.concatenate([Vp[kr, sl], ones], axis=1)
            o = _dot(e, av)
            outs.append(o[:, :_DH] * (1.0 / o[:, _DH:_DH + 1]))
        blocks.append(jnp.concatenate(outs, axis=-1))
    return jnp.concatenate(blocks, axis=0)


def _blk_front(dec_ref, exv_ref, x_ref, int_ref, ex_ref,
               mwq, mwk, mwv, mwo, g1, b1, cwq, cwk, cwv, cwo,
               x1_ref, qc_ref, exo_ref):
    base = pl.program_id(0) * _BB
    xf = x_ref[...].reshape(_BB * _Q, _D)
    y = _attn_block(_dot(xf, mwq[...]), _dot(xf, mwk[...]),
                    _dot(xf, mwv[...]), dec_ref, base, _Q, _Q)
    x1f = _layer_norm(xf + _dot(y, mwo[...]), g1[...], b1[...])
    x1_ref[...] = x1f.reshape(_BB, _Q, _D)
    qcf = jnp.concatenate(
        [x1f, int_ref[...].reshape(_BB * _Q, _DI)], axis=-1)
    qc_ref[...] = qcf.reshape(_BB, _Q, _D + _DI)
    exf = ex_ref[...].reshape(_BB * _EX, _D)
    co = _attn_block(_dot(qcf, cwq[...]), _dot(exf, cwk[...]),
                     _dot(exf, cwv[...]), exv_ref, base, _Q, _EX)
    exo_ref[...] = _dot(co, cwo[...]).reshape(_BB, _Q, _D)


def _blk_back(stat_ref, qc_ref, sk_ref, tk_ref, val_ref, x1_ref, exo_ref,
              wqs, wqt, wks, wkt, wv, wo,
              gt, w1, b1, w2, b2, g2, bb2, g3, bb3,
              out_ref, ss_s, ts_s):
    b = pl.program_id(0)
    par = lax.rem(b, 2)
    opar = 1 - par
    scale = 1.0 / math.sqrt(_D)

    ssT = ss_s[opar]
    swT = _softmax_ax(
        jnp.where(ssT >= _kth_threshold(ssT, _STAT_K, 0), ssT, _NEG), 0)
    ts3 = ts_s[opar].reshape(_S, _T, _Q)
    tw3 = _softmax_ax(
        jnp.where(ts3 >= _kth_threshold(ts3, _TOKEN_K, 1), ts3, _NEG), 1)
    comb = (swT[:, None, :] * tw3).reshape(_S * _T, _Q)
    ctx = lax.dot_general(comb, val_ref[0], (((0,), (0,)), ((), ())),
                          preferred_element_type=_F32)
    sel = _dot(_dot(ctx, wv[...]), wo[...])
    x1 = x1_ref[0]
    exo = exo_ref[0]
    gw = gt[...]
    logit = (jnp.sum(sel * gw[:, :_D], axis=-1, keepdims=True)
             + jnp.sum(exo * gw[:, _D:], axis=-1, keepdims=True))
    g = jax.nn.sigmoid(logit)
    x2 = _layer_norm(x1 + g * sel + (1.0 - g) * exo, g2[...], bb2[...])
    hh = jnp.maximum(_dot(x2, w1[...]) + b1[...], 0.0)
    ff = _dot(hh, w2[...]) + b2[...]
    out_ref[0] = _layer_norm(x2 + ff, g3[...], bb3[...])

    qc = qc_ref[0]
    qs = _dot(qc, wqs[...])
    ks = _dot(sk_ref[0], wks[...])
    sst = _dot_t(ks, qs) * scale
    gidx = lax.broadcasted_iota(jnp.int32, (_S, _Q), 0)
    ss_s[par] = jnp.where(gidx < stat_ref[jnp.minimum(b, _B - 1)], sst, _NEG)
    qt = _dot(qc, wqt[...])
    qt2 = _dot_t(qt, wkt[...])
    ts_s[par] = _dot_t(tk_ref[0], qt2) * scale


def _batched(shape, nb):
    n = len(shape) - 1
    return pl.BlockSpec((nb,) + tuple(shape[1:]),
                        lambda b, *_: (b,) + (0,) * n)


def _full(shape):
    n = len(shape)
    return pl.BlockSpec(tuple(shape), lambda b, *_: (0,) * n)


def _lead(shape):
    n = len(shape) - 1
    return pl.BlockSpec((1,) + tuple(shape[1:]),
                        lambda b, *_: (jnp.minimum(b, _B - 1),) + (0,) * n)


def _lag(shape):
    n = len(shape) - 1
    return pl.BlockSpec((1,) + tuple(shape[1:]),
                        lambda b, *_: (jnp.maximum(b - 1, 0),) + (0,) * n)


def _call(body, scalars, arrays, out_shapes, nb=1):
    in_specs = [_batched(a.shape, nb) if flag else _full(a.shape)
                for a, flag in arrays]
    grid_spec = pltpu.PrefetchScalarGridSpec(
        num_scalar_prefetch=len(scalars),
        grid=(_B // nb,),
        in_specs=in_specs,
        out_specs=tuple(_batched(s, nb) for s in out_shapes),
    )
    return pl.pallas_call(
        body,
        grid_spec=grid_spec,
        out_shape=tuple(jax.ShapeDtypeStruct(s, _F32) for s in out_shapes),
    )(*scalars, *(a for a, _ in arrays))


def kernel(x, intent, stat_keys, token_keys, values, exemplar, params,
           dec_valid_lens, stat_valid_lens, ex_valid_lens):
    P = params
    dec = dec_valid_lens.astype(jnp.int32)
    stv = stat_valid_lens.astype(jnp.int32)
    exv = ex_valid_lens.astype(jnp.int32)
    tk = token_keys.reshape(_B, _S * _T, _D)
    vals = values.reshape(_B, _S * _T, _D)
    r = lambda a, n: a.reshape(1, n)

    x1, qc, exo = _call(
        _blk_front, (dec, exv),
        [(x, True), (intent, True), (exemplar, True),
         (P['ma_Wq'], False), (P['ma_Wk'], False),
         (P['ma_Wv'], False), (P['ma_Wo'], False),
         (r(P['ln1_g'], _D), False), (r(P['ln1_b'], _D), False),
         (P['ca_Wq'], False), (P['ca_Wk'], False),
         (P['ca_Wv'], False), (P['ca_Wo'], False)],
        [(_B, _Q, _D), (_B, _Q, _D + _DI), (_B, _Q, _D)], nb=_BB)

    weights = [P['sa_Wqs'], P['sa_Wqt'], P['sa_Wks'], P['sa_Wkt'],
               P['sa_Wv'], P['sa_Wo'], P['gate_W'].reshape(1, 2 * _D),
               P['ffn_W1'], r(P['ffn_b1'], _DFF),
               P['ffn_W2'], r(P['ffn_b2'], _D),
               r(P['ln2_g'], _D), r(P['ln2_b'], _D),
               r(P['ln3_g'], _D), r(P['ln3_b'], _D)]
    grid_spec = pltpu.PrefetchScalarGridSpec(
        num_scalar_prefetch=1,
        grid=(_B + 1,),
        in_specs=[_lead(qc.shape), _lead(stat_keys.shape), _lead(tk.shape),
                  _lag(vals.shape), _lag(x1.shape), _lag(exo.shape),
                  *[_full(w.shape) for w in weights]],
        out_specs=(_lag((_B, _Q, _D)),),
        scratch_shapes=[pltpu.VMEM((2, _S, _Q), _F32),
                        pltpu.VMEM((2, _S * _T, _Q), _F32)],
    )
    (out,) = pl.pallas_call(
        _blk_back,
        grid_spec=grid_spec,
        out_shape=(jax.ShapeDtypeStruct((_B, _Q, _D), _F32),),
    )(stv, qc, stat_keys, tk, vals, x1, exo, *weights)
    return out
